# baseline (speedup 1.0000x reference)
import jax
import jax.numpy as jnp
from jax.experimental import pallas as pl

_RADIUS = 0.2
_NSAMPLE = 32


def _ball_query(xyz, new_xyz, radius, nsample):
    N = xyz.shape[1]
    d2 = (jnp.sum(new_xyz ** 2, axis=-1)[:, :, None]
          + jnp.sum(xyz ** 2, axis=-1)[:, None, :]
          - 2.0 * jnp.einsum('bpd,bnd->bpn', new_xyz, xyz))
    mask = d2 < radius * radius
    arange = jnp.arange(N, dtype=jnp.int32)[None, None, :]
    scores = jnp.where(mask, arange, jnp.int32(N))
    sorted_idx = jnp.sort(scores, axis=-1)[..., :nsample]
    first = sorted_idx[..., :1]
    idx = jnp.where(sorted_idx >= N, first, sorted_idx)
    idx = jnp.where(idx >= N, 0, idx)
    return idx


def _group(feats, idx):
    return jax.vmap(lambda f, i: f[:, i])(feats, idx)


def _copy_kernel(x_ref, o_ref):
    o_ref[...] = x_ref[...]


def kernel(xyz, new_xyz, features):
    idx = _ball_query(xyz, new_xyz, _RADIUS, _NSAMPLE)
    xyz_trans = jnp.transpose(xyz, (0, 2, 1))
    grouped_xyz = _group(xyz_trans, idx)
    grouped_xyz = grouped_xyz - jnp.transpose(new_xyz, (0, 2, 1))[..., None]
    grouped_features = _group(features, idx)
    new_features = jnp.concatenate([grouped_xyz, grouped_features], axis=1)
    B, C, P, S = new_features.shape
    out = pl.pallas_call(
        _copy_kernel,
        grid=(B, C),
        in_specs=[pl.BlockSpec((1, 1, P, S), lambda b, c: (b, c, 0, 0))],
        out_specs=pl.BlockSpec((1, 1, P, S), lambda b, c: (b, c, 0, 0)),
        out_shape=jax.ShapeDtypeStruct(new_features.shape, new_features.dtype),
    )(new_features)
    return out


# speedup vs baseline: 2.3228x; 2.3228x over previous
import jax
import jax.numpy as jnp
from jax.experimental import pallas as pl

_QB = 256
_N = 8192
_NS = 32
_R2 = 0.2 * 0.2


def _bq_kernel(q_ref, x_ref, o_ref):
    q = q_ref[0]
    x = x_ref[0]
    qb = q.astype(jnp.bfloat16)
    xb = x.astype(jnp.bfloat16)
    mm = jax.lax.dot_general(qb, xb, (((1,), (1,)), ((), ())),
                             preferred_element_type=jnp.float32)
    sq_q = (q[:, 0] * q[:, 0] + q[:, 1] * q[:, 1]) + q[:, 2] * q[:, 2]
    sq_x = (x[:, 0] * x[:, 0] + x[:, 1] * x[:, 1]) + x[:, 2] * x[:, 2]
    d2 = (sq_q[:, None] + sq_x[None, :]) - 2.0 * mm
    m = jnp.where(d2 < _R2, 1.0, 0.0)
    r = m
    k = 1
    while k < _N:
        r = r + jnp.concatenate(
            [jnp.zeros((_QB, k), jnp.float32), r[:, :_N - k]], axis=1)
        k *= 2
    cols = []
    for s in range(_NS):
        cs = jnp.sum(jnp.where(r <= float(s), 1.0, 0.0), axis=1)
        cols.append(cs[:, None])
    c = jnp.concatenate(cols, axis=1)
    fallback = jnp.where(c[:, 0:1] < float(_N), c[:, 0:1], 0.0)
    idx = jnp.where(c < float(_N), c, fallback).astype(jnp.int32)
    o_ref[0] = idx


def _ball_query(xyz, new_xyz):
    B, P, _ = new_xyz.shape
    return pl.pallas_call(
        _bq_kernel,
        grid=(B, P // _QB),
        in_specs=[pl.BlockSpec((1, _QB, 3), lambda b, p: (b, p, 0)),
                  pl.BlockSpec((1, _N, 3), lambda b, p: (b, 0, 0))],
        out_specs=pl.BlockSpec((1, _QB, _NS), lambda b, p: (b, p, 0)),
        out_shape=jax.ShapeDtypeStruct((B, P, _NS), jnp.int32),
    )(new_xyz, xyz)


def kernel(xyz, new_xyz, features):
    idx = _ball_query(xyz, new_xyz)
    xyz_trans = jnp.transpose(xyz, (0, 2, 1))
    grouped_xyz = jax.vmap(lambda f, i: f[:, i])(xyz_trans, idx)
    grouped_xyz = grouped_xyz - jnp.transpose(new_xyz, (0, 2, 1))[..., None]
    grouped_features = jax.vmap(lambda f, i: f[:, i])(features, idx)
    return jnp.concatenate([grouped_xyz, grouped_features], axis=1)


# speedup vs baseline: 9.4424x; 4.0650x over previous
import functools

import jax
import jax.numpy as jnp
from jax import lax
from jax.experimental import pallas as pl
from jax.experimental.pallas import tpu as pltpu
from jax.experimental.pallas import tpu_sc as plsc

_QB = 256
_N = 8192
_NS = 32
_P = 1024
_B = 4
_C = 64
_ROWS = _B * (_C + 3)
_FLAT = _P * _NS
_R2 = 0.2 * 0.2
_NWORKERS = 32
_NC = 2


def _bq_kernel(q_ref, x_ref, o_ref):
    q = q_ref[0]
    x = x_ref[0]
    qb = q.astype(jnp.bfloat16)
    xb = x.astype(jnp.bfloat16)
    mm = lax.dot_general(qb, xb, (((1,), (1,)), ((), ())),
                         preferred_element_type=jnp.float32)
    sq_q = (q[:, 0] * q[:, 0] + q[:, 1] * q[:, 1]) + q[:, 2] * q[:, 2]
    sq_x = (x[:, 0] * x[:, 0] + x[:, 1] * x[:, 1]) + x[:, 2] * x[:, 2]
    d2 = (sq_q[:, None] + sq_x[None, :]) - 2.0 * mm
    m = jnp.where(d2 < _R2, 1.0, 0.0)
    r = m
    k = 1
    while k < _N:
        r = r + jnp.concatenate(
            [jnp.zeros((_QB, k), jnp.float32), r[:, :_N - k]], axis=1)
        k *= 2
    cols = []
    for s in range(_NS):
        cs = jnp.sum(jnp.where(r <= float(s), 1.0, 0.0), axis=1)
        cols.append(cs[:, None])
    c = jnp.concatenate(cols, axis=1)
    fallback = jnp.where(c[:, 0:1] < float(_N), c[:, 0:1], 0.0)
    idx = jnp.where(c < float(_N), c, fallback).astype(jnp.int32)
    o_ref[0] = idx


def _ball_query(xyz, new_xyz):
    return pl.pallas_call(
        _bq_kernel,
        grid=(_B, _P // _QB),
        in_specs=[pl.BlockSpec((1, _QB, 3), lambda b, p: (b, p, 0)),
                  pl.BlockSpec((1, _N, 3), lambda b, p: (b, 0, 0))],
        out_specs=pl.BlockSpec((1, _QB, _NS), lambda b, p: (b, p, 0)),
        out_shape=jax.ShapeDtypeStruct((_B, _P, _NS), jnp.int32),
    )(new_xyz, xyz)


def _sc_group_body(feats_hbm, idx_hbm, q_hbm, out_hbm,
                   row_v, idx_v, out_v, q_v):
    cid = lax.axis_index("c")
    sid = lax.axis_index("s")
    w = sid * _NC + cid
    lo = (w * _ROWS) // _NWORKERS
    hi = ((w + 1) * _ROWS) // _NWORKERS

    def gather_row(i, _):
        iv = idx_v[pl.ds(i * 16, 16)]
        vals = plsc.load_gather(row_v, [iv])
        qi = jnp.full((16,), i // 2, jnp.int32)
        qs = plsc.load_gather(q_v, [qi])
        out_v[pl.ds(i * 16, 16)] = vals - qs
        return 0

    prev_b = jnp.int32(-1)
    for j in range(9):
        r = lo + j

        @pl.when(r < hi)
        def _do(r=r, prev_b=prev_b):
            b = r // (_C + 3)
            ch = lax.rem(r, _C + 3)

            @pl.when(b != prev_b)
            def _load_idx():
                pltpu.sync_copy(idx_hbm.at[b], idx_v)

            pltpu.sync_copy(feats_hbm.at[b, ch], row_v)
            pltpu.sync_copy(q_hbm.at[b, ch], q_v)
            lax.fori_loop(0, _FLAT // 16, gather_row, 0)
            pltpu.sync_copy(out_v, out_hbm.at[r])

        prev_b = jnp.where(r < hi, r // (_C + 3), prev_b)


@functools.partial(
    pl.kernel,
    out_type=jax.ShapeDtypeStruct((_ROWS, _FLAT), jnp.float32),
    mesh=plsc.VectorSubcoreMesh(core_axis_name="c", subcore_axis_name="s"),
    compiler_params=pltpu.CompilerParams(use_tc_tiling_on_sc=False,
                                         needs_layout_passes=False),
    scratch_types=[
        pltpu.VMEM((_N,), jnp.float32),
        pltpu.VMEM((_FLAT,), jnp.int32),
        pltpu.VMEM((_FLAT,), jnp.float32),
        pltpu.VMEM((_P,), jnp.float32),
    ],
)
def _sc_group(feats_hbm, idx_hbm, q_hbm, out_hbm, row_v, idx_v, out_v, q_v):
    _sc_group_body(feats_hbm, idx_hbm, q_hbm, out_hbm,
                   row_v, idx_v, out_v, q_v)


def kernel(xyz, new_xyz, features):
    idx = _ball_query(xyz, new_xyz)
    xyz_t = jnp.transpose(xyz, (0, 2, 1))
    feats_all = jnp.concatenate([xyz_t, features], axis=1)
    q_t = jnp.transpose(new_xyz, (0, 2, 1))
    q_all = jnp.concatenate(
        [q_t, jnp.zeros((_B, _C, _P), jnp.float32)], axis=1)
    out = _sc_group(feats_all, idx.reshape(_B, _FLAT), q_all)
    return out.reshape(_B, _C + 3, _P, _NS)


# speedup vs baseline: 23.0706x; 2.4433x over previous
import functools

import jax
import jax.numpy as jnp
from jax import lax
from jax.experimental import pallas as pl
from jax.experimental.pallas import tpu as pltpu
from jax.experimental.pallas import tpu_sc as plsc

_QB = 256
_N = 8192
_NS = 32
_P = 1024
_B = 4
_C = 64
_ROWS = _B * (_C + 3)
_FLAT = _P * _NS
_R2 = 0.2 * 0.2
_NW = 32
_NC = 2
_W = _N // 16
_NQ = _B * _P
_QPW = _NQ // _NW

_SC_PARAMS = pltpu.CompilerParams(use_tc_tiling_on_sc=False,
                                  needs_layout_passes=False)


def _bq_pack_kernel(q_ref, x_ref, p_ref, o_ref):
    q = q_ref[0]
    x = x_ref[0]
    qb = q.astype(jnp.bfloat16)
    xb = x.astype(jnp.bfloat16)
    mm = lax.dot_general(qb, xb, (((1,), (1,)), ((), ())),
                         preferred_element_type=jnp.float32)
    sq_q = (q[:, 0] * q[:, 0] + q[:, 1] * q[:, 1]) + q[:, 2] * q[:, 2]
    sq_x = (x[:, 0] * x[:, 0] + x[:, 1] * x[:, 1]) + x[:, 2] * x[:, 2]
    d2 = (sq_q[:, None] + sq_x[None, :]) - 2.0 * mm
    m = jnp.where(d2 < _R2, 1.0, 0.0).astype(jnp.bfloat16)
    bits = lax.dot_general(m, p_ref[...], (((1,), (0,)), ((), ())),
                           preferred_element_type=jnp.float32)
    o_ref[0] = bits.astype(jnp.int32)


def _pack_matrix():
    n = jnp.arange(_N)
    w = 16 * (n // 256) + (n % 16)
    j = (n % 256) // 16
    onehot = (w[:, None] == jnp.arange(_W)[None, :])
    return jnp.where(onehot, (2.0 ** j)[:, None], 0.0).astype(jnp.bfloat16)


def _ball_query_bits(xyz, new_xyz):
    return pl.pallas_call(
        _bq_pack_kernel,
        grid=(_B, _P // _QB),
        in_specs=[pl.BlockSpec((1, _QB, 3), lambda b, p: (b, p, 0)),
                  pl.BlockSpec((1, _N, 3), lambda b, p: (b, 0, 0)),
                  pl.BlockSpec((_N, _W), lambda b, p: (0, 0))],
        out_specs=pl.BlockSpec((1, _QB, _W), lambda b, p: (b, p, 0)),
        out_shape=jax.ShapeDtypeStruct((_B, _P, _W), jnp.int32),
    )(new_xyz, xyz, _pack_matrix())


def _sc_select_body(bits_hbm, idx_hbm, bits_v, idx_v, buf_v):
    cid = lax.axis_index("c")
    sid = lax.axis_index("s")
    w = sid * _NC + cid
    q0 = w * _QPW
    pltpu.sync_copy(bits_hbm.at[pl.ds(q0, _QPW)], bits_v)
    iota = lax.iota(jnp.int32, 16)

    def one_query(qi, _):
        def cond(carry):
            v, cnt = carry
            return jnp.logical_and(cnt < _NS, v < _W // 16)

        def body(carry):
            v, cnt = carry
            w16 = bits_v[qi, pl.ds(v * 16, 16)]
            base = iota + v * 256
            for j in range(16):
                mj = ((w16 >> j) & 1) == 1
                vals = base + j * 16
                plsc.store_compressed(
                    buf_v.at[pl.ds(jnp.minimum(cnt, _NS), 16)], vals, mask=mj)
                cnt = cnt + jnp.sum(mj.astype(jnp.int32))
            return (v + 1, cnt)

        _, cnt = lax.while_loop(cond, body, (jnp.int32(0), jnp.int32(0)))

        cnt_v = jnp.full((16,), cnt, jnp.int32)
        first = plsc.load_gather(buf_v, [jnp.zeros((16,), jnp.int32)])
        first = jnp.where(cnt_v > 0, first, jnp.zeros((16,), jnp.int32))
        lo = buf_v[pl.ds(0, 16)]
        hi = buf_v[pl.ds(16, 16)]
        idx_v[qi, pl.ds(0, 16)] = jnp.where(iota < cnt_v, lo, first)
        idx_v[qi, pl.ds(16, 16)] = jnp.where(iota + 16 < cnt_v, hi, first)
        return 0

    lax.fori_loop(0, _QPW, one_query, 0)
    pltpu.sync_copy(idx_v, idx_hbm.at[pl.ds(q0, _QPW)])


@functools.partial(
    pl.kernel,
    out_type=jax.ShapeDtypeStruct((_NQ, _NS), jnp.int32),
    mesh=plsc.VectorSubcoreMesh(core_axis_name="c", subcore_axis_name="s"),
    compiler_params=_SC_PARAMS,
    scratch_types=[
        pltpu.VMEM((_QPW, _W), jnp.int32),
        pltpu.VMEM((_QPW, _NS), jnp.int32),
        pltpu.VMEM((64,), jnp.int32),
    ],
)
def _sc_select(bits_hbm, idx_hbm, bits_v, idx_v, buf_v):
    _sc_select_body(bits_hbm, idx_hbm, bits_v, idx_v, buf_v)


def _sc_group_body(xyzt_hbm, feat_hbm, idx_hbm, q_hbm, out_hbm,
                   row_v, idx_v, out_v, q_v):
    cid = lax.axis_index("c")
    sid = lax.axis_index("s")
    w = sid * _NC + cid
    lo = (w * _ROWS) // _NW
    hi = ((w + 1) * _ROWS) // _NW

    def gather_plain(i, _):
        base = i * 64
        for u in range(4):
            iv = idx_v[pl.ds(base + u * 16, 16)]
            out_v[pl.ds(base + u * 16, 16)] = plsc.load_gather(row_v, [iv])
        return 0

    def gather_sub(i, _):
        base = i * 64
        for u in range(4):
            iv = idx_v[pl.ds(base + u * 16, 16)]
            vals = plsc.load_gather(row_v, [iv])
            qi = jnp.full((16,), (base + u * 16) // 32, jnp.int32)
            qs = plsc.load_gather(q_v, [qi])
            out_v[pl.ds(base + u * 16, 16)] = vals - qs
        return 0

    prev_b = jnp.int32(-1)
    for j in range(9):
        r = lo + j

        @pl.when(r < hi)
        def _do(r=r, prev_b=prev_b):
            b = r // (_C + 3)
            ch = lax.rem(r, _C + 3)

            @pl.when(b != prev_b)
            def _load_idx():
                pltpu.sync_copy(idx_hbm.at[b], idx_v)

            @pl.when(ch < 3)
            def _xyz_row():
                pltpu.sync_copy(xyzt_hbm.at[b, ch], row_v)
                pltpu.sync_copy(q_hbm.at[b, ch], q_v)
                lax.fori_loop(0, _FLAT // 64, gather_sub, 0)

            @pl.when(ch >= 3)
            def _feat_row():
                pltpu.sync_copy(feat_hbm.at[b, ch - 3], row_v)
                lax.fori_loop(0, _FLAT // 64, gather_plain, 0)

            pltpu.sync_copy(out_v, out_hbm.at[r])

        prev_b = jnp.where(r < hi, r // (_C + 3), prev_b)


@functools.partial(
    pl.kernel,
    out_type=jax.ShapeDtypeStruct((_ROWS, _FLAT), jnp.float32),
    mesh=plsc.VectorSubcoreMesh(core_axis_name="c", subcore_axis_name="s"),
    compiler_params=_SC_PARAMS,
    scratch_types=[
        pltpu.VMEM((_N,), jnp.float32),
        pltpu.VMEM((_FLAT,), jnp.int32),
        pltpu.VMEM((_FLAT,), jnp.float32),
        pltpu.VMEM((_P,), jnp.float32),
    ],
)
def _sc_group(xyzt_hbm, feat_hbm, idx_hbm, q_hbm, out_hbm,
              row_v, idx_v, out_v, q_v):
    _sc_group_body(xyzt_hbm, feat_hbm, idx_hbm, q_hbm, out_hbm,
                   row_v, idx_v, out_v, q_v)


def kernel(xyz, new_xyz, features):
    bits = _ball_query_bits(xyz, new_xyz)
    idx = _sc_select(bits.reshape(_NQ, _W))
    xyz_t = jnp.transpose(xyz, (0, 2, 1))
    q_t = jnp.transpose(new_xyz, (0, 2, 1))
    out = _sc_group(xyz_t, features, idx.reshape(_B, _FLAT), q_t)
    return out.reshape(_B, _C + 3, _P, _NS)
